# baseline (speedup 1.0000x reference)
import functools

import jax
import jax.numpy as jnp
from jax import lax
from jax.experimental import pallas as pl
from jax.experimental.pallas import tpu as pltpu
from jax.experimental.pallas import tpu_sc as plsc

N_PTS = 100000
N_VERTS = 6890
N_JOINTS = 24

NPAD = 102400
VPAD = 6912
PBLK = 512
PBLK2 = 2048
GW = 32

NW = 32
BPW = NPAD // NW
GRP = 128
NGRP = BPW // GRP

_NEG = -3.0e38


def _knn_kernel(xyz_ref, vt_ref, sw_ref, bt_ref, verts_ref,
                i1_ref, i2_ref, g_ref):
    x = xyz_ref[...]
    vt = vt_ref[...]
    xh = jnp.concatenate([x, jnp.ones((PBLK, 1), jnp.float32)], axis=1)
    vsq = vt[0:1] * vt[0:1] + vt[1:2] * vt[1:2] + vt[2:3] * vt[2:3]
    vext = jnp.concatenate([vt, -0.5 * vsq], axis=0)
    s = jnp.dot(xh, vext, preferred_element_type=jnp.float32,
                precision=lax.Precision.HIGHEST)
    lane = lax.broadcasted_iota(jnp.int32, (PBLK, VPAD), 1)
    s = jnp.where(lane >= N_VERTS, _NEG, s)

    m1 = jnp.max(s, axis=1, keepdims=True)
    i1 = jnp.min(jnp.where(s == m1, lane, VPAD), axis=1)
    s2 = jnp.where(lane == i1[:, None], _NEG, s)
    m2 = jnp.max(s2, axis=1, keepdims=True)
    i2 = jnp.min(jnp.where(s2 == m2, lane, VPAD), axis=1)
    i1_ref[...] = i1
    i2_ref[...] = i2

    @pl.when(pl.program_id(0) == 0)
    def _build_table():
        t_all = jnp.dot(sw_ref[...], bt_ref[...],
                        preferred_element_type=jnp.float32,
                        precision=lax.Precision.HIGHEST)
        idxf = lax.broadcasted_iota(jnp.int32, (N_VERTS, 1), 0)
        g_ref[...] = jnp.concatenate(
            [t_all, verts_ref[...], idxf.astype(jnp.float32),
             jnp.zeros((N_VERTS, GW - 20), jnp.float32)], axis=1)


def _knn_call(xyz_pad, verts_t, sw, bt16, verts):
    return pl.pallas_call(
        _knn_kernel,
        grid=(NPAD // PBLK,),
        in_specs=[
            pl.BlockSpec((PBLK, 3), lambda i: (i, 0)),
            pl.BlockSpec((3, VPAD), lambda i: (0, 0)),
            pl.BlockSpec((N_VERTS, N_JOINTS), lambda i: (0, 0)),
            pl.BlockSpec((N_JOINTS, 16), lambda i: (0, 0)),
            pl.BlockSpec((N_VERTS, 3), lambda i: (0, 0)),
        ],
        out_specs=[
            pl.BlockSpec((PBLK,), lambda i: (i,)),
            pl.BlockSpec((PBLK,), lambda i: (i,)),
            pl.BlockSpec((N_VERTS, GW), lambda i: (0, 0)),
        ],
        out_shape=[
            jax.ShapeDtypeStruct((NPAD,), jnp.int32),
            jax.ShapeDtypeStruct((NPAD,), jnp.int32),
            jax.ShapeDtypeStruct((N_VERTS, GW), jnp.float32),
        ],
    )(xyz_pad, verts_t, sw, bt16, verts)


def _sc_gather(g, i1, i2):
    mesh = plsc.VectorSubcoreMesh(core_axis_name="c", subcore_axis_name="s")

    @functools.partial(
        pl.kernel, mesh=mesh,
        compiler_params=pltpu.CompilerParams(use_tc_tiling_on_sc=False),
        out_type=[jax.ShapeDtypeStruct((NPAD, GW), jnp.float32),
                  jax.ShapeDtypeStruct((NPAD, GW), jnp.float32)],
        scratch_types=[
            pltpu.VMEM((BPW,), jnp.int32),
            pltpu.VMEM((BPW,), jnp.int32),
            pltpu.VMEM((GRP, GW), jnp.float32),
            pltpu.VMEM((GRP, GW), jnp.float32),
            pltpu.SemaphoreType.DMA,
            pltpu.SemaphoreType.DMA,
        ],
    )
    def body(g_hbm, i1_hbm, i2_hbm, o1_hbm, o2_hbm,
             idx1_v, idx2_v, rows1_v, rows2_v, sem1, sem2):
        wid = lax.axis_index("s") * 2 + lax.axis_index("c")
        base = pl.multiple_of(wid * BPW, GRP)
        pltpu.sync_copy(i1_hbm.at[pl.ds(base, BPW)], idx1_v)
        pltpu.sync_copy(i2_hbm.at[pl.ds(base, BPW)], idx2_v)

        def step(j, carry):
            off = pl.multiple_of(j * GRP, GRP)
            dst = pl.multiple_of(base + j * GRP, GRP)
            c1 = pltpu.async_copy(g_hbm.at[idx1_v.at[pl.ds(off, GRP)]],
                                  rows1_v, sem1)
            c2 = pltpu.async_copy(g_hbm.at[idx2_v.at[pl.ds(off, GRP)]],
                                  rows2_v, sem2)
            c1.wait()
            c2.wait()
            pltpu.sync_copy(rows1_v, o1_hbm.at[pl.ds(dst, GRP)])
            pltpu.sync_copy(rows2_v, o2_hbm.at[pl.ds(dst, GRP)])
            return carry

        lax.fori_loop(0, NGRP, step, 0)

    return body(g, i1, i2)


def _finish_kernel(xyz_ref, rot_ref, g1_ref, g2_ref, xb_ref, rb_ref):
    x = xyz_ref[:, 0:1]
    y = xyz_ref[:, 1:2]
    z = xyz_ref[:, 2:3]
    g1 = g1_ref[...]
    g2 = g2_ref[...]

    def dist(g):
        dx = x - g[:, 16:17]
        dy = y - g[:, 17:18]
        dz = z - g[:, 18:19]
        return dx * dx + dy * dy + dz * dz

    d1 = dist(g1)
    d2 = dist(g2)
    id1 = g1[:, 19:20]
    id2 = g2[:, 19:20]
    pick1 = (d1 < d2) | ((d1 == d2) & (id1 <= id2))
    t = jnp.where(pick1, g1[:, 0:16], g2[:, 0:16])
    tc = [t[:, k:k + 1] for k in range(16)]

    xb0 = tc[0] * x + tc[1] * y + tc[2] * z + tc[3]
    xb1 = tc[4] * x + tc[5] * y + tc[6] * z + tc[7]
    xb2 = tc[8] * x + tc[9] * y + tc[10] * z + tc[11]
    xb_ref[...] = jnp.concatenate([xb0, xb1, xb2], axis=1)

    r0 = rot_ref[:, 0:1]
    r1 = rot_ref[:, 1:2]
    r2 = rot_ref[:, 2:3]
    r3 = rot_ref[:, 3:4]
    nrm = jnp.sqrt(r0 * r0 + r1 * r1 + r2 * r2 + r3 * r3)
    qw = r0 / nrm
    qx = r1 / nrm
    qy = r2 / nrm
    qz = r3 / nrm
    R = [
        1 - 2 * (qy * qy + qz * qz), 2 * (qx * qy - qw * qz), 2 * (qx * qz + qw * qy),
        2 * (qx * qy + qw * qz), 1 - 2 * (qx * qx + qz * qz), 2 * (qy * qz - qw * qx),
        2 * (qx * qz - qw * qy), 2 * (qy * qz + qw * qx), 1 - 2 * (qx * qx + qy * qy),
    ]
    rb = []
    for rr in range(3):
        for cc in range(3):
            rb.append(tc[4 * rr] * R[cc] + tc[4 * rr + 1] * R[3 + cc]
                      + tc[4 * rr + 2] * R[6 + cc])
    rb_ref[...] = jnp.concatenate(rb, axis=1)


def _finish_call(xyz_pad, rot_pad, g1, g2):
    return pl.pallas_call(
        _finish_kernel,
        grid=(NPAD // PBLK2,),
        in_specs=[
            pl.BlockSpec((PBLK2, 3), lambda i: (i, 0)),
            pl.BlockSpec((PBLK2, 4), lambda i: (i, 0)),
            pl.BlockSpec((PBLK2, GW), lambda i: (i, 0)),
            pl.BlockSpec((PBLK2, GW), lambda i: (i, 0)),
        ],
        out_specs=[
            pl.BlockSpec((PBLK2, 3), lambda i: (i, 0)),
            pl.BlockSpec((PBLK2, 9), lambda i: (i, 0)),
        ],
        out_shape=[
            jax.ShapeDtypeStruct((NPAD, 3), jnp.float32),
            jax.ShapeDtypeStruct((NPAD, 9), jnp.float32),
        ],
    )(xyz_pad, rot_pad, g1, g2)


def kernel(xyz, rotation, bone_transforms, smpl_verts, skinning_weights):
    n = xyz.shape[0]
    xyz_pad = jnp.pad(xyz, ((0, NPAD - n), (0, 0)))
    rot_pad = jnp.pad(rotation, ((0, NPAD - n), (0, 0)), constant_values=1.0)
    verts_t = jnp.pad(smpl_verts.T, ((0, 0), (0, VPAD - N_VERTS)))
    bt16 = bone_transforms.reshape(N_JOINTS, 16)

    i1, i2, g = _knn_call(xyz_pad, verts_t, skinning_weights, bt16, smpl_verts)
    g1, g2 = _sc_gather(g, i1, i2)
    xb, rb = _finish_call(xyz_pad, rot_pad, g1, g2)
    return xb[:n], rb[:n].reshape(n, 3, 3)


# speedup vs baseline: 1.6510x; 1.6510x over previous
import functools

import jax
import jax.numpy as jnp
from jax import lax
from jax.experimental import pallas as pl
from jax.experimental.pallas import tpu as pltpu
from jax.experimental.pallas import tpu_sc as plsc

N_PTS = 100000
N_VERTS = 6890
N_JOINTS = 24

NPAD = 102400
VPAD = 6912
PBLK = 1024
PBLK2 = 2048
GW = 32

NW = 32
BPW = NPAD // NW
GRP = 128
NGRP = BPW // GRP

_NEG = -3.0e38


def _knn_kernel(xyz_ref, vt_ref, sw_ref, bt_ref, verts_ref,
                i1_ref, i2_ref, g_ref):
    x = xyz_ref[...]
    vt = vt_ref[...]
    xh = jnp.concatenate([x, jnp.ones((PBLK, 1), jnp.float32)], axis=1)
    vsq = vt[0:1] * vt[0:1] + vt[1:2] * vt[1:2] + vt[2:3] * vt[2:3]
    vext = jnp.concatenate([vt, -0.5 * vsq], axis=0)

    s = jnp.dot(xh, vext, preferred_element_type=jnp.float32,
                precision=lax.Precision.HIGHEST)

    lane = lax.broadcasted_iota(jnp.int32, (PBLK, 128), 1)
    m1 = jnp.full((PBLK, 128), _NEG, jnp.float32)
    m2 = jnp.full((PBLK, 128), _NEG, jnp.float32)
    t1 = jnp.zeros((PBLK, 128), jnp.int32)
    t2 = jnp.zeros((PBLK, 128), jnp.int32)
    nslab = VPAD // 128
    for t in range(nslab):
        st = s[:, 128 * t:128 * (t + 1)]
        if (t + 1) * 128 > N_VERTS:
            st = jnp.where(lane >= N_VERTS - 128 * t, _NEG, st)
        g1 = st > m1
        g2 = st > m2
        m2 = jnp.where(g1, m1, jnp.where(g2, st, m2))
        t2 = jnp.where(g1, t1, jnp.where(g2, t, t2))
        m1 = jnp.where(g1, st, m1)
        t1 = jnp.where(g1, t, t1)

    def first_lane(vals, mx):
        return jnp.min(jnp.where(vals == mx, lane, 128), axis=1,
                       keepdims=True)

    def extract_i(arr, oh):
        return jnp.sum(jnp.where(oh, arr, 0), axis=1, keepdims=True)

    mx1 = jnp.max(m1, axis=1, keepdims=True)
    l1 = first_lane(m1, mx1)
    oh1 = lane == l1
    m1m = jnp.where(oh1, _NEG, m1)
    mx1b = jnp.max(m1m, axis=1, keepdims=True)
    l2 = first_lane(m1m, mx1b)
    oh2 = lane == l2
    val_a = extract_i(m2, oh1)
    slab_a = extract_i(t2, oh1)
    slab_b = extract_i(t1, oh2)
    c1 = extract_i(t1, oh1) * 128 + l1
    use_a = val_a >= mx1b
    c2 = jnp.where(use_a, slab_a * 128 + l1, slab_b * 128 + l2)
    i1_ref[...] = c1[:, 0]
    i2_ref[...] = c2[:, 0]

    @pl.when(pl.program_id(0) == 0)
    def _build_table():
        t_all = jnp.dot(sw_ref[...], bt_ref[...],
                        preferred_element_type=jnp.float32,
                        precision=lax.Precision.HIGHEST)
        idxf = lax.broadcasted_iota(jnp.int32, (N_VERTS, 1), 0)
        g_ref[...] = jnp.concatenate(
            [t_all, verts_ref[...], idxf.astype(jnp.float32),
             jnp.zeros((N_VERTS, GW - 20), jnp.float32)], axis=1)


def _knn_call(xyz_pad, verts_t, sw, bt16, verts):
    return pl.pallas_call(
        _knn_kernel,
        grid=(NPAD // PBLK,),
        in_specs=[
            pl.BlockSpec((PBLK, 3), lambda i: (i, 0)),
            pl.BlockSpec((3, VPAD), lambda i: (0, 0)),
            pl.BlockSpec((N_VERTS, N_JOINTS), lambda i: (0, 0)),
            pl.BlockSpec((N_JOINTS, 16), lambda i: (0, 0)),
            pl.BlockSpec((N_VERTS, 3), lambda i: (0, 0)),
        ],
        out_specs=[
            pl.BlockSpec((PBLK,), lambda i: (i,)),
            pl.BlockSpec((PBLK,), lambda i: (i,)),
            pl.BlockSpec((N_VERTS, GW), lambda i: (0, 0)),
        ],
        out_shape=[
            jax.ShapeDtypeStruct((NPAD,), jnp.int32),
            jax.ShapeDtypeStruct((NPAD,), jnp.int32),
            jax.ShapeDtypeStruct((N_VERTS, GW), jnp.float32),
        ],
    )(xyz_pad, verts_t, sw, bt16, verts)


def _sc_gather(g, i1, i2):
    mesh = plsc.VectorSubcoreMesh(core_axis_name="c", subcore_axis_name="s")

    @functools.partial(
        pl.kernel, mesh=mesh,
        compiler_params=pltpu.CompilerParams(use_tc_tiling_on_sc=False),
        out_type=[jax.ShapeDtypeStruct((NPAD, GW), jnp.float32),
                  jax.ShapeDtypeStruct((NPAD, GW), jnp.float32)],
        scratch_types=[
            pltpu.VMEM((BPW,), jnp.int32),
            pltpu.VMEM((BPW,), jnp.int32),
            pltpu.VMEM((GRP, GW), jnp.float32),
            pltpu.VMEM((GRP, GW), jnp.float32),
            pltpu.SemaphoreType.DMA,
            pltpu.SemaphoreType.DMA,
        ],
    )
    def body(g_hbm, i1_hbm, i2_hbm, o1_hbm, o2_hbm,
             idx1_v, idx2_v, rows1_v, rows2_v, sem1, sem2):
        wid = lax.axis_index("s") * 2 + lax.axis_index("c")
        base = pl.multiple_of(wid * BPW, GRP)
        pltpu.sync_copy(i1_hbm.at[pl.ds(base, BPW)], idx1_v)
        pltpu.sync_copy(i2_hbm.at[pl.ds(base, BPW)], idx2_v)

        def step(j, carry):
            off = pl.multiple_of(j * GRP, GRP)
            dst = pl.multiple_of(base + j * GRP, GRP)
            c1 = pltpu.async_copy(g_hbm.at[idx1_v.at[pl.ds(off, GRP)]],
                                  rows1_v, sem1)
            c2 = pltpu.async_copy(g_hbm.at[idx2_v.at[pl.ds(off, GRP)]],
                                  rows2_v, sem2)
            c1.wait()
            c2.wait()
            pltpu.sync_copy(rows1_v, o1_hbm.at[pl.ds(dst, GRP)])
            pltpu.sync_copy(rows2_v, o2_hbm.at[pl.ds(dst, GRP)])
            return carry

        lax.fori_loop(0, NGRP, step, 0)

    return body(g, i1, i2)


def _finish_kernel(xyz_ref, rot_ref, g1_ref, g2_ref, xb_ref, rb_ref):
    x = xyz_ref[0:1, :]
    y = xyz_ref[1:2, :]
    z = xyz_ref[2:3, :]
    g1 = g1_ref[...]
    g2 = g2_ref[...]

    def dist(g):
        dx = x - g[16:17, :]
        dy = y - g[17:18, :]
        dz = z - g[18:19, :]
        return dx * dx + dy * dy + dz * dz

    d1 = dist(g1)
    d2 = dist(g2)
    id1 = g1[19:20, :]
    id2 = g2[19:20, :]
    pick1 = (d1 < d2) | ((d1 == d2) & (id1 <= id2))
    tc = [jnp.where(pick1, g1[k:k + 1, :], g2[k:k + 1, :]) for k in range(16)]

    xb0 = tc[0] * x + tc[1] * y + tc[2] * z + tc[3]
    xb1 = tc[4] * x + tc[5] * y + tc[6] * z + tc[7]
    xb2 = tc[8] * x + tc[9] * y + tc[10] * z + tc[11]
    xb_ref[...] = jnp.concatenate([xb0, xb1, xb2], axis=0)

    r0 = rot_ref[0:1, :]
    r1 = rot_ref[1:2, :]
    r2 = rot_ref[2:3, :]
    r3 = rot_ref[3:4, :]
    nrm = jnp.sqrt(r0 * r0 + r1 * r1 + r2 * r2 + r3 * r3)
    qw = r0 / nrm
    qx = r1 / nrm
    qy = r2 / nrm
    qz = r3 / nrm
    R = [
        1 - 2 * (qy * qy + qz * qz), 2 * (qx * qy - qw * qz), 2 * (qx * qz + qw * qy),
        2 * (qx * qy + qw * qz), 1 - 2 * (qx * qx + qz * qz), 2 * (qy * qz - qw * qx),
        2 * (qx * qz - qw * qy), 2 * (qy * qz + qw * qx), 1 - 2 * (qx * qx + qy * qy),
    ]
    rb = []
    for rr in range(3):
        for cc in range(3):
            rb.append(tc[4 * rr] * R[cc] + tc[4 * rr + 1] * R[3 + cc]
                      + tc[4 * rr + 2] * R[6 + cc])
    rb_ref[...] = jnp.concatenate(rb, axis=0)


LBLK = 6400


def _finish_call(xyz_t, rot_t, g1t, g2t):
    return pl.pallas_call(
        _finish_kernel,
        grid=(NPAD // LBLK,),
        in_specs=[
            pl.BlockSpec((3, LBLK), lambda i: (0, i)),
            pl.BlockSpec((4, LBLK), lambda i: (0, i)),
            pl.BlockSpec((GW, LBLK), lambda i: (0, i)),
            pl.BlockSpec((GW, LBLK), lambda i: (0, i)),
        ],
        out_specs=[
            pl.BlockSpec((3, LBLK), lambda i: (0, i)),
            pl.BlockSpec((9, LBLK), lambda i: (0, i)),
        ],
        out_shape=[
            jax.ShapeDtypeStruct((3, NPAD), jnp.float32),
            jax.ShapeDtypeStruct((9, NPAD), jnp.float32),
        ],
    )(xyz_t, rot_t, g1t, g2t)


def kernel(xyz, rotation, bone_transforms, smpl_verts, skinning_weights):
    n = xyz.shape[0]
    xyz_pad = jnp.pad(xyz, ((0, NPAD - n), (0, 0)))
    rot_pad = jnp.pad(rotation, ((0, NPAD - n), (0, 0)), constant_values=1.0)
    verts_t = jnp.pad(smpl_verts.T, ((0, 0), (0, VPAD - N_VERTS)))
    bt16 = bone_transforms.reshape(N_JOINTS, 16)

    i1, i2, g = _knn_call(xyz_pad, verts_t, skinning_weights, bt16, smpl_verts)
    g1, g2 = _sc_gather(g, i1, i2)
    xb_t, rb_t = _finish_call(xyz_pad.T, rot_pad.T, g1.T, g2.T)
    return xb_t.T[:n], rb_t.T[:n].reshape(n, 3, 3)
